# Initial kernel scaffold; baseline (speedup 1.0000x reference)
import jax
import jax.numpy as jnp
from jax.experimental import pallas as pl


def kernel(inputs, weight, bias):
    raise NotImplementedError("write your pallas kernel here")


# speedup vs baseline: 17.7009x; 17.7009x over previous
import functools

import jax
import jax.numpy as jnp
from jax import lax
from jax.experimental import pallas as pl
from jax.experimental.pallas import tpu as pltpu
from jax.experimental.pallas import tpu_sc as plsc

M, K, N = 8192, 4096, 4096
BM, BN, BK = 1024, 1024, 512

NTOT = N * K
NW = 32
ROW = NTOT // NW
CHUNK = 16384
NCH = ROW // CHUNK
NBINS = 4096
RANK = 15099494
S1 = float(2 ** 18)
W1 = float(2 ** -18)
S2 = float(2 ** 30)
W2 = float(2 ** -30)


def _zero_hist(hist):
    zeros = jnp.zeros((16,), jnp.int32)

    def zb(i, _):
        hist[pl.ds(i * 16, 16)] = zeros
        return 0

    lax.fori_loop(0, (NBINS * 16) // 16, zb, 0)


def _lane_reduce_and_store(hist, histout, out_hbm, wid):
    def rb(v, _):
        acc = hist[pl.ds(v * 16, 16)]
        for l in range(1, 16):
            acc = acc + hist[pl.ds(l * NBINS + v * 16, 16)]
        histout[pl.ds(v * 16, 16)] = acc
        return 0

    lax.fori_loop(0, NBINS // 16, rb, 0)
    pltpu.sync_copy(histout, out_hbm.at[pl.ds(wid * NBINS, NBINS)])


def _pass1_body(w_hbm, out_hbm, data, hist, histout):
    wid = lax.axis_index("s") * 2 + lax.axis_index("c")
    _zero_hist(hist)
    lane_off = lax.iota(jnp.int32, 16) * NBINS
    ones = jnp.ones((16,), jnp.int32)

    def chunk_body(ch, _):
        pltpu.sync_copy(w_hbm.at[pl.ds(wid * ROW + ch * CHUNK, CHUNK)], data)

        def vec_body(i, _):
            a = jnp.abs(data[pl.ds(i * 16, 16)])
            ib = jnp.clip((a * S1).astype(jnp.int32), 0, NBINS - 1)
            plsc.addupdate_scatter(hist, [ib + lane_off], ones)
            return 0

        lax.fori_loop(0, CHUNK // 16, vec_body, 0)
        return 0

    lax.fori_loop(0, NCH, chunk_body, 0)
    _lane_reduce_and_store(hist, histout, out_hbm, wid)


def _pass2_body(w_hbm, seli_hbm, self_hbm, out_hbm, data, hist, histout,
                scal_i, scal_f):
    wid = lax.axis_index("s") * 2 + lax.axis_index("c")
    _zero_hist(hist)
    pltpu.sync_copy(seli_hbm, scal_i)
    pltpu.sync_copy(self_hbm, scal_f)
    b1v = scal_i[...]
    lov = scal_f[...]
    lane_off = lax.iota(jnp.int32, 16) * NBINS
    ones = jnp.ones((16,), jnp.int32)

    def chunk_body(ch, _):
        pltpu.sync_copy(w_hbm.at[pl.ds(wid * ROW + ch * CHUNK, CHUNK)], data)

        def vec_body(i, _):
            a = jnp.abs(data[pl.ds(i * 16, 16)])
            ib = jnp.clip((a * S1).astype(jnp.int32), 0, NBINS - 1)
            fine = jnp.clip(((a - lov) * S2).astype(jnp.int32), 0, NBINS - 1)
            plsc.addupdate_scatter(hist, [fine + lane_off], ones,
                                   mask=ib == b1v)
            return 0

        lax.fori_loop(0, CHUNK // 16, vec_body, 0)
        return 0

    lax.fori_loop(0, NCH, chunk_body, 0)
    _lane_reduce_and_store(hist, histout, out_hbm, wid)


def _sc_mesh():
    return plsc.VectorSubcoreMesh(core_axis_name="c", subcore_axis_name="s")


def _sc_pass1(wflat):
    return pl.kernel(
        _pass1_body,
        out_type=jax.ShapeDtypeStruct((NW * NBINS,), jnp.int32),
        mesh=_sc_mesh(),
        scratch_types=[
            pltpu.VMEM((CHUNK,), jnp.float32),
            pltpu.VMEM((NBINS * 16,), jnp.int32),
            pltpu.VMEM((NBINS,), jnp.int32),
        ],
        compiler_params=pltpu.CompilerParams(needs_layout_passes=False),
    )(wflat)


def _sc_pass2(wflat, seli, self_):
    return pl.kernel(
        _pass2_body,
        out_type=jax.ShapeDtypeStruct((NW * NBINS,), jnp.int32),
        mesh=_sc_mesh(),
        scratch_types=[
            pltpu.VMEM((CHUNK,), jnp.float32),
            pltpu.VMEM((NBINS * 16,), jnp.int32),
            pltpu.VMEM((NBINS,), jnp.int32),
            pltpu.VMEM((16,), jnp.int32),
            pltpu.VMEM((16,), jnp.float32),
        ],
        compiler_params=pltpu.CompilerParams(needs_layout_passes=False),
    )(wflat, seli, self_)


def _cumsum_lanes(c):
    s = 1
    while s < NBINS:
        z = jnp.zeros((1, s), jnp.int32)
        c = c + jnp.concatenate([z, c[:, : NBINS - s]], axis=1)
        s *= 2
    return c


def _select1_kernel(h_ref, oi_ref, of_ref, or_ref):
    tot = jnp.sum(h_ref[...], axis=0, keepdims=True)
    c = _cumsum_lanes(tot)
    le = c <= RANK
    b1 = jnp.sum(le.astype(jnp.int32))
    base = jnp.max(jnp.where(le, c, 0))
    oi_ref[...] = jnp.full((1, 16), b1, jnp.int32)
    of_ref[...] = jnp.full((1, 16), b1.astype(jnp.float32) * W1, jnp.float32)
    or_ref[...] = jnp.full((1, 16), RANK - base, jnp.int32)


def _select1(h):
    return pl.pallas_call(
        _select1_kernel,
        out_shape=[
            jax.ShapeDtypeStruct((1, 16), jnp.int32),
            jax.ShapeDtypeStruct((1, 16), jnp.float32),
            jax.ShapeDtypeStruct((1, 16), jnp.int32),
        ],
    )(h)


def _select2_kernel(h_ref, r2_ref, lo_ref, t_ref):
    tot = jnp.sum(h_ref[...], axis=0, keepdims=True)
    c = _cumsum_lanes(tot)
    b2 = jnp.sum((c <= r2_ref[0, 0]).astype(jnp.int32))
    t_ref[0, 0] = lo_ref[0, 0] + b2.astype(jnp.float32) * W2


def _select2(h2, r2, lo):
    return pl.pallas_call(
        _select2_kernel,
        out_specs=pl.BlockSpec(memory_space=pltpu.SMEM),
        out_shape=jax.ShapeDtypeStruct((1, 1), jnp.float32),
    )(h2, r2, lo)


def _mm_body(thr_ref, x_ref, w_ref, b_ref, o_ref):
    k = pl.program_id(2)
    t = thr_ref[0, 0]
    w = w_ref[...]
    w = jnp.where(jnp.abs(w) >= t, w, 0.0)
    acc = jax.lax.dot_general(
        x_ref[...], w, (((1,), (1,)), ((), ())),
        preferred_element_type=jnp.float32,
    )

    @pl.when(k == 0)
    def _():
        o_ref[...] = acc + b_ref[...]

    @pl.when(k != 0)
    def _():
        o_ref[...] += acc


def _masked_matmul(threshold, inputs, weight, bias2d):
    grid = (M // BM, N // BN, K // BK)
    return pl.pallas_call(
        _mm_body,
        grid=grid,
        in_specs=[
            pl.BlockSpec(memory_space=pltpu.SMEM),
            pl.BlockSpec((BM, BK), lambda m, n, k: (m, k)),
            pl.BlockSpec((BN, BK), lambda m, n, k: (n, k)),
            pl.BlockSpec((1, BN), lambda m, n, k: (0, n)),
        ],
        out_specs=pl.BlockSpec((BM, BN), lambda m, n, k: (m, n)),
        out_shape=jax.ShapeDtypeStruct((M, N), jnp.float32),
    )(threshold, inputs, weight, bias2d)


@jax.jit
def kernel(inputs, weight, bias):
    wflat = weight.reshape(NTOT)
    h1 = _sc_pass1(wflat).reshape(NW, NBINS)
    b1, lo, r2 = _select1(h1)
    h2 = _sc_pass2(wflat, b1.reshape(16), lo.reshape(16)).reshape(NW, NBINS)
    t = _select2(h2, r2, lo)
    return _masked_matmul(t, inputs, weight, bias.reshape(1, N))


# speedup vs baseline: 20.7717x; 1.1735x over previous
import functools

import jax
import jax.numpy as jnp
from jax import lax
from jax.experimental import pallas as pl
from jax.experimental.pallas import tpu as pltpu
from jax.experimental.pallas import tpu_sc as plsc

M, K, N = 8192, 4096, 4096
BM, BN, BK = 1024, 1024, 1024

NTOT = N * K
NW = 32
ROW = NTOT // NW
CHUNK = 16384
NCH = ROW // CHUNK
UNROLL = 4
NBINS = 4096
RANK = 15099494
S1 = float(2 ** 18)
W1 = float(2 ** -18)
S2 = float(2 ** 30)
W2 = float(2 ** -30)


def _zero_hist(hist):
    zeros = jnp.zeros((16,), jnp.int32)

    def zb(i, _):
        hist[pl.ds(i * 16, 16)] = zeros
        return 0

    lax.fori_loop(0, (NBINS * 16) // 16, zb, 0)


def _lane_reduce_and_store(hist, histout, out_hbm, wid):
    def rb(v, _):
        acc = hist[pl.ds(v * 16, 16)]
        for l in range(1, 16):
            acc = acc + hist[pl.ds(l * NBINS + v * 16, 16)]
        histout[pl.ds(v * 16, 16)] = acc
        return 0

    lax.fori_loop(0, NBINS // 16, rb, 0)
    pltpu.sync_copy(histout, out_hbm.at[pl.ds(wid * NBINS, NBINS)])


def _chunked_scan(w_hbm, data0, data1, sem0, sem1, wid, process_vec):
    base = wid * ROW
    pltpu.async_copy(w_hbm.at[pl.ds(base, CHUNK)], data0, sem0)
    pltpu.async_copy(w_hbm.at[pl.ds(base + CHUNK, CHUNK)], data1, sem1)

    def process_chunk(data):
        def vec_body(v, _):
            for j in range(UNROLL):
                process_vec(data[pl.ds((v * UNROLL + j) * 16, 16)])
            return 0

        lax.fori_loop(0, CHUNK // (16 * UNROLL), vec_body, 0)

    def body(i, _):
        pltpu.make_async_copy(w_hbm.at[pl.ds(0, CHUNK)], data0, sem0).wait()
        process_chunk(data0)

        @pl.when(i < NCH // 2 - 1)
        def _():
            pltpu.async_copy(
                w_hbm.at[pl.ds(base + (2 * i + 2) * CHUNK, CHUNK)], data0, sem0)

        pltpu.make_async_copy(w_hbm.at[pl.ds(0, CHUNK)], data1, sem1).wait()
        process_chunk(data1)

        @pl.when(i < NCH // 2 - 1)
        def _():
            pltpu.async_copy(
                w_hbm.at[pl.ds(base + (2 * i + 3) * CHUNK, CHUNK)], data1, sem1)

        return 0

    lax.fori_loop(0, NCH // 2, body, 0)


def _pass1_body(w_hbm, out_hbm, data0, data1, hist, histout, sem0, sem1):
    wid = lax.axis_index("s") * 2 + lax.axis_index("c")
    _zero_hist(hist)
    lane_off = lax.iota(jnp.int32, 16) * NBINS
    ones = jnp.ones((16,), jnp.int32)
    top = jnp.full((16,), NBINS - 1, jnp.int32)

    def process_vec(x):
        a = jnp.abs(x)
        ib = jnp.minimum((a * S1).astype(jnp.int32), top)
        plsc.addupdate_scatter(hist, [ib + lane_off], ones)

    _chunked_scan(w_hbm, data0, data1, sem0, sem1, wid, process_vec)
    _lane_reduce_and_store(hist, histout, out_hbm, wid)


def _pass2_body(w_hbm, seli_hbm, self_hbm, out_hbm, data0, data1, hist,
                histout, scal_i, scal_f, sem0, sem1):
    wid = lax.axis_index("s") * 2 + lax.axis_index("c")
    _zero_hist(hist)
    pltpu.sync_copy(seli_hbm, scal_i)
    pltpu.sync_copy(self_hbm, scal_f)
    b1v = scal_i[...]
    lov = scal_f[...]
    lane_off = lax.iota(jnp.int32, 16) * NBINS
    ones = jnp.ones((16,), jnp.int32)
    top = jnp.full((16,), NBINS - 1, jnp.int32)

    def process_vec(x):
        a = jnp.abs(x)
        ib = jnp.minimum((a * S1).astype(jnp.int32), top)
        fine = jnp.minimum(((a - lov) * S2).astype(jnp.int32), top)
        plsc.addupdate_scatter(hist, [fine + lane_off], ones, mask=ib == b1v)

    _chunked_scan(w_hbm, data0, data1, sem0, sem1, wid, process_vec)
    _lane_reduce_and_store(hist, histout, out_hbm, wid)


def _sc_mesh():
    return plsc.VectorSubcoreMesh(core_axis_name="c", subcore_axis_name="s")


def _sc_pass1(wflat):
    return pl.kernel(
        _pass1_body,
        out_type=jax.ShapeDtypeStruct((NW * NBINS,), jnp.int32),
        mesh=_sc_mesh(),
        scratch_types=[
            pltpu.VMEM((CHUNK,), jnp.float32),
            pltpu.VMEM((CHUNK,), jnp.float32),
            pltpu.VMEM((NBINS * 16,), jnp.int32),
            pltpu.VMEM((NBINS,), jnp.int32),
            pltpu.SemaphoreType.DMA,
            pltpu.SemaphoreType.DMA,
        ],
        compiler_params=pltpu.CompilerParams(needs_layout_passes=False),
    )(wflat)


def _sc_pass2(wflat, seli, self_):
    return pl.kernel(
        _pass2_body,
        out_type=jax.ShapeDtypeStruct((NW * NBINS,), jnp.int32),
        mesh=_sc_mesh(),
        scratch_types=[
            pltpu.VMEM((CHUNK,), jnp.float32),
            pltpu.VMEM((CHUNK,), jnp.float32),
            pltpu.VMEM((NBINS * 16,), jnp.int32),
            pltpu.VMEM((NBINS,), jnp.int32),
            pltpu.VMEM((16,), jnp.int32),
            pltpu.VMEM((16,), jnp.float32),
            pltpu.SemaphoreType.DMA,
            pltpu.SemaphoreType.DMA,
        ],
        compiler_params=pltpu.CompilerParams(needs_layout_passes=False),
    )(wflat, seli, self_)


def _cumsum_lanes(c):
    s = 1
    while s < NBINS:
        z = jnp.zeros((1, s), jnp.int32)
        c = c + jnp.concatenate([z, c[:, : NBINS - s]], axis=1)
        s *= 2
    return c


def _select1_kernel(h_ref, oi_ref, of_ref, or_ref):
    tot = jnp.sum(h_ref[...], axis=0, keepdims=True)
    c = _cumsum_lanes(tot)
    le = c <= RANK
    b1 = jnp.sum(le.astype(jnp.int32))
    base = jnp.max(jnp.where(le, c, 0))
    oi_ref[...] = jnp.full((1, 16), b1, jnp.int32)
    of_ref[...] = jnp.full((1, 16), b1.astype(jnp.float32) * W1, jnp.float32)
    or_ref[...] = jnp.full((1, 16), RANK - base, jnp.int32)


def _select1(h):
    return pl.pallas_call(
        _select1_kernel,
        out_shape=[
            jax.ShapeDtypeStruct((1, 16), jnp.int32),
            jax.ShapeDtypeStruct((1, 16), jnp.float32),
            jax.ShapeDtypeStruct((1, 16), jnp.int32),
        ],
    )(h)


def _select2_kernel(h_ref, r2_ref, lo_ref, t_ref):
    tot = jnp.sum(h_ref[...], axis=0, keepdims=True)
    c = _cumsum_lanes(tot)
    b2 = jnp.sum((c <= r2_ref[0, 0]).astype(jnp.int32))
    t_ref[0, 0] = lo_ref[0, 0] + b2.astype(jnp.float32) * W2


def _select2(h2, r2, lo):
    return pl.pallas_call(
        _select2_kernel,
        out_specs=pl.BlockSpec(memory_space=pltpu.SMEM),
        out_shape=jax.ShapeDtypeStruct((1, 1), jnp.float32),
    )(h2, r2, lo)


PBM = 512


def _premask_body(thr_ref, w_ref, o_ref):
    t = thr_ref[0, 0]
    w = w_ref[...]
    o_ref[...] = jnp.where(jnp.abs(w) >= t, w, 0.0).astype(jnp.bfloat16)


def _premask(threshold, weight):
    return pl.pallas_call(
        _premask_body,
        grid=(N // PBM,),
        in_specs=[
            pl.BlockSpec(memory_space=pltpu.SMEM),
            pl.BlockSpec((PBM, K), lambda i: (i, 0)),
        ],
        out_specs=pl.BlockSpec((PBM, K), lambda i: (i, 0)),
        out_shape=jax.ShapeDtypeStruct((N, K), jnp.bfloat16),
    )(threshold, weight)


def _mm_body(x_ref, w_ref, b_ref, o_ref):
    k = pl.program_id(2)
    acc = jax.lax.dot_general(
        x_ref[...].astype(jnp.bfloat16), w_ref[...], (((1,), (1,)), ((), ())),
        preferred_element_type=jnp.float32,
    )

    @pl.when(k == 0)
    def _():
        o_ref[...] = acc + b_ref[...]

    @pl.when(k != 0)
    def _():
        o_ref[...] += acc


def _masked_matmul(threshold, inputs, weight, bias2d):
    wm = _premask(threshold, weight)
    grid = (M // BM, N // BN, K // BK)
    return pl.pallas_call(
        _mm_body,
        grid=grid,
        in_specs=[
            pl.BlockSpec((BM, BK), lambda m, n, k: (m, k)),
            pl.BlockSpec((BN, BK), lambda m, n, k: (n, k)),
            pl.BlockSpec((1, BN), lambda m, n, k: (0, n)),
        ],
        out_specs=pl.BlockSpec((BM, BN), lambda m, n, k: (m, n)),
        out_shape=jax.ShapeDtypeStruct((M, N), jnp.float32),
    )(inputs, wm, bias2d)


@jax.jit
def kernel(inputs, weight, bias):
    wflat = weight.reshape(NTOT)
    h1 = _sc_pass1(wflat).reshape(NW, NBINS)
    b1, lo, r2 = _select1(h1)
    h2 = _sc_pass2(wflat, b1.reshape(16), lo.reshape(16)).reshape(NW, NBINS)
    t = _select2(h2, r2, lo)
    return _masked_matmul(t, inputs, weight, bias.reshape(1, N))
